# Initial kernel scaffold; baseline (speedup 1.0000x reference)
import jax
import jax.numpy as jnp
from jax import lax
from jax.experimental import pallas as pl
from jax.experimental.pallas import tpu as pltpu
from jax.experimental.pallas import tpu_sc as plsc

B, S, H = 4, 2048, 768
V, T = 100000, 2
N = B * S
NC, NS, L = 2, 16, 16
NW = NC * NS
TOK_PER_W = N // NW
CH = 64
NCH = TOK_PER_W // CH
NLG = H // L


def _emb_body(ids_hbm, tt_hbm, word_hbm, type_hbm, pos_hbm, out_hbm,
              idx_v, ttv, wbuf, pbuf, t0v, t1v, sem):
    cid = lax.axis_index("c")
    sid = lax.axis_index("s")
    wid = sid * NC + cid
    base = wid * TOK_PER_W
    pos_base = base % S

    pltpu.sync_copy(type_hbm.at[0], t0v)
    pltpu.sync_copy(type_hbm.at[1], t1v)

    for ch in range(NCH):
        tb = base + ch * CH
        pltpu.sync_copy(ids_hbm.at[pl.ds(tb, CH)], idx_v)
        gather = pltpu.async_copy(word_hbm.at[idx_v], wbuf, sem)
        pltpu.sync_copy(tt_hbm.at[pl.ds(tb, CH)], ttv)
        pltpu.sync_copy(pos_hbm.at[pl.ds(pos_base + ch * CH, CH)], pbuf)
        gather.wait()

        for l in range(NLG):
            sl = pl.ds(l * L, L)
            t0 = t0v[sl]
            dv = t1v[sl] - t0

            def tok_body(t, carry, sl=sl, t0=t0, dv=dv):
                ttf = ttv[t].astype(jnp.float32)
                wbuf[t, sl] = wbuf[t, sl] + pbuf[t, sl] + t0 + ttf * dv
                return carry

            lax.fori_loop(0, CH, tok_body, 0)

        pltpu.sync_copy(wbuf, out_hbm.at[pl.ds(tb, CH)])


@jax.jit
def _emb_call(ids_flat, tt_flat, word_emb, type_emb, pos_emb):
    mesh = plsc.VectorSubcoreMesh(core_axis_name="c", subcore_axis_name="s")
    fn = pl.kernel(
        _emb_body,
        out_type=jax.ShapeDtypeStruct((N, H), jnp.float32),
        mesh=mesh,
        scratch_types=[
            pltpu.VMEM((CH,), jnp.int32),
            pltpu.VMEM((CH,), jnp.int32),
            pltpu.VMEM((CH, H), jnp.float32),
            pltpu.VMEM((CH, H), jnp.float32),
            pltpu.VMEM((H,), jnp.float32),
            pltpu.VMEM((H,), jnp.float32),
            pltpu.SemaphoreType.DMA,
        ],
    )
    return fn(ids_flat, tt_flat, word_emb, type_emb, pos_emb)


def kernel(input_ids, token_type_ids, word_emb, type_emb, pos_emb):
    ids_flat = input_ids.reshape(-1).astype(jnp.int32)
    tt_flat = token_type_ids.reshape(-1).astype(jnp.int32)
    out = _emb_call(ids_flat, tt_flat, word_emb, type_emb, pos_emb)
    return out.reshape(B, S, H)


# speedup vs baseline: 1.5346x; 1.5346x over previous
import jax
import jax.numpy as jnp
from jax import lax
from jax.experimental import pallas as pl
from jax.experimental.pallas import tpu as pltpu
from jax.experimental.pallas import tpu_sc as plsc

B, S, H = 4, 2048, 768
V, T = 100000, 2
N = B * S
NC, NS, L = 2, 16, 16
NW = NC * NS
TOK_PER_W = N // NW
CH = 64
NCH = TOK_PER_W // CH
NLG = H // L


def _emb_body(ids_hbm, tt_hbm, word_hbm, type_hbm, pos_hbm, out_hbm,
              idx_v, ttv, wbuf, pbuf, t0v, t1v, dvv, sem):
    cid = lax.axis_index("c")
    sid = lax.axis_index("s")
    wid = sid * NC + cid
    base = wid * TOK_PER_W
    pos_base = base % S

    pltpu.sync_copy(type_hbm.at[0], t0v)
    pltpu.sync_copy(type_hbm.at[1], t1v)
    for l in range(NLG):
        sl = pl.ds(l * L, L)
        dvv[sl] = t1v[sl] - t0v[sl]

    for ch in range(NCH):
        tb = base + ch * CH
        pltpu.sync_copy(ids_hbm.at[pl.ds(tb, CH)], idx_v)
        gather = pltpu.async_copy(word_hbm.at[idx_v], wbuf, sem)
        pltpu.sync_copy(tt_hbm.at[pl.ds(tb, CH)], ttv)
        pltpu.sync_copy(pos_hbm.at[pl.ds(pos_base + ch * CH, CH)], pbuf)
        gather.wait()

        def grp_body(g, carry):
            ttgf = ttv[pl.ds(g * L, L)].astype(jnp.float32)

            def lg_body(l, c2):
                sl = pl.ds(l * L, L)
                t0 = t0v[sl]
                dv = dvv[sl]
                for j in range(L):
                    t = g * L + j
                    wbuf[t, sl] = (wbuf[t, sl] + pbuf[t, sl]) + (t0 + ttgf[j] * dv)
                return c2

            lax.fori_loop(0, NLG, lg_body, 0)
            return carry

        lax.fori_loop(0, CH // L, grp_body, 0)

        pltpu.sync_copy(wbuf, out_hbm.at[pl.ds(tb, CH)])


@jax.jit
def _emb_call(ids_flat, tt_flat, word_emb, type_emb, pos_emb):
    mesh = plsc.VectorSubcoreMesh(core_axis_name="c", subcore_axis_name="s")
    fn = pl.kernel(
        _emb_body,
        out_type=jax.ShapeDtypeStruct((N, H), jnp.float32),
        mesh=mesh,
        scratch_types=[
            pltpu.VMEM((CH,), jnp.int32),
            pltpu.VMEM((CH,), jnp.int32),
            pltpu.VMEM((CH, H), jnp.float32),
            pltpu.VMEM((CH, H), jnp.float32),
            pltpu.VMEM((H,), jnp.float32),
            pltpu.VMEM((H,), jnp.float32),
            pltpu.VMEM((H,), jnp.float32),
            pltpu.SemaphoreType.DMA,
        ],
    )
    return fn(ids_flat, tt_flat, word_emb, type_emb, pos_emb)


def kernel(input_ids, token_type_ids, word_emb, type_emb, pos_emb):
    ids_flat = input_ids.reshape(-1).astype(jnp.int32)
    tt_flat = token_type_ids.reshape(-1).astype(jnp.int32)
    out = _emb_call(ids_flat, tt_flat, word_emb, type_emb, pos_emb)
    return out.reshape(B, S, H)


# speedup vs baseline: 2.0043x; 1.3061x over previous
import jax
import jax.numpy as jnp
from jax import lax
from jax.experimental import pallas as pl
from jax.experimental.pallas import tpu as pltpu
from jax.experimental.pallas import tpu_sc as plsc

B, S, H = 4, 2048, 768
V, T = 100000, 2
N = B * S
NC, NS, L = 2, 16, 16
NW = NC * NS
PB = 64
CH = 32
NLG = H // L


def _emb_body(ids_hbm, tt_hbm, word_hbm, type_hbm, pos_hbm, out_hbm,
              idxv0, idxv1, ttv0, ttv1, wbuf0, wbuf1, posb, t0v, t1v, dvv,
              gs0, gs1, os0, os1):
    cid = lax.axis_index("c")
    sid = lax.axis_index("s")
    wid = sid * NC + cid
    pbase = wid * PB

    pltpu.sync_copy(pos_hbm.at[pl.ds(pbase, PB)], posb)
    pltpu.sync_copy(type_hbm.at[0], t0v)
    pltpu.sync_copy(type_hbm.at[1], t1v)
    for l in range(NLG):
        sl = pl.ds(l * L, L)
        dvv[sl] = t1v[sl] - t0v[sl]

    idxv = [idxv0, idxv1]
    ttv = [ttv0, ttv1]
    wbuf = [wbuf0, wbuf1]
    gsem = [gs0, gs1]
    osem = [os0, os1]

    chunks = [(b, h) for b in range(B) for h in range(PB // CH)]

    def tbase(b, h):
        return b * S + pbase + h * CH

    b0, h0 = chunks[0]
    pltpu.sync_copy(ids_hbm.at[pl.ds(tbase(b0, h0), CH)], idxv[0])
    ghandles = [pltpu.async_copy(word_hbm.at[idxv[0]], wbuf[0], gsem[0]), None]
    out_handles = [None, None]

    for c, (b, h) in enumerate(chunks):
        p = c & 1
        tb = tbase(b, h)
        if c + 1 < len(chunks):
            bn, hn = chunks[c + 1]
            if out_handles[1 - p] is not None:
                out_handles[1 - p].wait()
            pltpu.sync_copy(ids_hbm.at[pl.ds(tbase(bn, hn), CH)], idxv[1 - p])
            ghandles[1 - p] = pltpu.async_copy(
                word_hbm.at[idxv[1 - p]], wbuf[1 - p], gsem[1 - p])
        pltpu.sync_copy(tt_hbm.at[pl.ds(tb, CH)], ttv[p])
        ghandles[p].wait()

        wb = wbuf[p]
        tv = ttv[p]

        def l_body(l, carry, wb=wb, tv=tv, h=h):
            sl = pl.ds(l * L, L)
            t0 = t0v[sl]
            dv = dvv[sl]
            for g in range(CH // L):
                ttfv = tv[pl.ds(g * L, L)].astype(jnp.float32)
                for j in range(L):
                    t = g * L + j
                    wb[t, sl] = wb[t, sl] + posb[h * CH + t, sl] + (t0 + ttfv[j] * dv)
            return carry

        lax.fori_loop(0, NLG, l_body, 0)

        out_handles[p] = pltpu.async_copy(wb, out_hbm.at[pl.ds(tb, CH)], osem[p])

    out_handles[0].wait()
    out_handles[1].wait()


@jax.jit
def _emb_call(ids_flat, tt_flat, word_emb, type_emb, pos_emb):
    mesh = plsc.VectorSubcoreMesh(core_axis_name="c", subcore_axis_name="s")
    fn = pl.kernel(
        _emb_body,
        out_type=jax.ShapeDtypeStruct((N, H), jnp.float32),
        mesh=mesh,
        scratch_types=[
            pltpu.VMEM((CH,), jnp.int32),
            pltpu.VMEM((CH,), jnp.int32),
            pltpu.VMEM((CH,), jnp.int32),
            pltpu.VMEM((CH,), jnp.int32),
            pltpu.VMEM((CH, H), jnp.float32),
            pltpu.VMEM((CH, H), jnp.float32),
            pltpu.VMEM((PB, H), jnp.float32),
            pltpu.VMEM((H,), jnp.float32),
            pltpu.VMEM((H,), jnp.float32),
            pltpu.VMEM((H,), jnp.float32),
            pltpu.SemaphoreType.DMA,
            pltpu.SemaphoreType.DMA,
            pltpu.SemaphoreType.DMA,
            pltpu.SemaphoreType.DMA,
        ],
    )
    return fn(ids_flat, tt_flat, word_emb, type_emb, pos_emb)


def kernel(input_ids, token_type_ids, word_emb, type_emb, pos_emb):
    ids_flat = input_ids.reshape(-1).astype(jnp.int32)
    tt_flat = token_type_ids.reshape(-1).astype(jnp.int32)
    out = _emb_call(ids_flat, tt_flat, word_emb, type_emb, pos_emb)
    return out.reshape(B, S, H)


# speedup vs baseline: 2.5025x; 1.2486x over previous
import jax
import jax.numpy as jnp
from jax import lax
from jax.experimental import pallas as pl
from jax.experimental.pallas import tpu as pltpu
from jax.experimental.pallas import tpu_sc as plsc

B, S, H = 4, 2048, 768
V, T = 100000, 2
N = B * S
NC, NS, L = 2, 16, 16
NW = NC * NS
PB = 64
PH = 16
BP = 2
CH = PH * BP
NCHUNK = (PB // PH) * (B // BP)
NLG = H // L


def _emb_body(ids_hbm, tt_hbm, word_hbm, type_hbm, pos_hbm, out_hbm,
              idtmp, tttmp, idxall, ttall, wbuf0, wbuf1, posb, t0v, t1v, dvv,
              psem, gs0, gs1, os0, os1):
    cid = lax.axis_index("c")
    sid = lax.axis_index("s")
    wid = sid * NC + cid
    pbase = wid * PB

    handles = []
    for b in range(B):
        handles.append(pltpu.async_copy(
            ids_hbm.at[pl.ds(b * S + pbase, PB)], idtmp.at[b], psem))
        handles.append(pltpu.async_copy(
            tt_hbm.at[pl.ds(b * S + pbase, PB)], tttmp.at[b], psem))
    handles.append(pltpu.async_copy(pos_hbm.at[pl.ds(pbase, PB)], posb, psem))
    handles.append(pltpu.async_copy(type_hbm.at[0], t0v, psem))
    handles.append(pltpu.async_copy(type_hbm.at[1], t1v, psem))
    for cp in handles:
        cp.wait()

    chunks = [(h, bp) for h in range(PB // PH) for bp in range(B // BP)]
    for c, (h, bp) in enumerate(chunks):
        for b2 in range(BP):
            b = bp * BP + b2
            dst = pl.ds(c * CH + b2 * PH, PH)
            idxall[dst] = idtmp[b, pl.ds(h * PH, PH)]
            ttall[dst] = tttmp[b, pl.ds(h * PH, PH)]

    wbuf = [wbuf0, wbuf1]
    gsem = [gs0, gs1]
    osem = [os0, os1]

    ghandles = [
        pltpu.async_copy(word_hbm.at[idxall.at[pl.ds(0, CH)]], wbuf[0], gsem[0]),
        None,
    ]
    out_handles = [None, None]

    for l in range(NLG):
        sl = pl.ds(l * L, L)
        dvv[sl] = t1v[sl] - t0v[sl]

    for c, (h, bp) in enumerate(chunks):
        p = c & 1
        if c + 1 < NCHUNK:
            if out_handles[1 - p] is not None:
                for oh in out_handles[1 - p]:
                    oh.wait()
            ghandles[1 - p] = pltpu.async_copy(
                word_hbm.at[idxall.at[pl.ds((c + 1) * CH, CH)]],
                wbuf[1 - p], gsem[1 - p])
        ghandles[p].wait()

        wb = wbuf[p]
        ttf = [ttall[pl.ds(c * CH + b2 * PH, PH)].astype(jnp.float32)
               for b2 in range(BP)]

        def l_body(l, carry, wb=wb, ttf=ttf, h=h):
            sl = pl.ds(l * L, L)
            t0 = t0v[sl]
            dv = dvv[sl]
            for r in range(PH):
                pp = posb[h * PH + r, sl] + t0
                for b2 in range(BP):
                    t = b2 * PH + r
                    wb[t, sl] = wb[t, sl] + pp + ttf[b2][r] * dv
            return carry

        lax.fori_loop(0, NLG, l_body, 0)

        out_handles[p] = []
        for b2 in range(BP):
            b = bp * BP + b2
            row0 = b * S + pbase + h * PH
            out_handles[p].append(pltpu.async_copy(
                wb.at[pl.ds(b2 * PH, PH)],
                out_hbm.at[pl.ds(row0, PH)], osem[p]))

    for hs in out_handles:
        if hs is not None:
            for oh in hs:
                oh.wait()


@jax.jit
def _emb_call(ids_flat, tt_flat, word_emb, type_emb, pos_emb):
    mesh = plsc.VectorSubcoreMesh(core_axis_name="c", subcore_axis_name="s")
    fn = pl.kernel(
        _emb_body,
        out_type=jax.ShapeDtypeStruct((N, H), jnp.float32),
        mesh=mesh,
        scratch_types=[
            pltpu.VMEM((B, PB), jnp.int32),
            pltpu.VMEM((B, PB), jnp.int32),
            pltpu.VMEM((N // NW,), jnp.int32),
            pltpu.VMEM((N // NW,), jnp.int32),
            pltpu.VMEM((CH, H), jnp.float32),
            pltpu.VMEM((CH, H), jnp.float32),
            pltpu.VMEM((PB, H), jnp.float32),
            pltpu.VMEM((H,), jnp.float32),
            pltpu.VMEM((H,), jnp.float32),
            pltpu.VMEM((H,), jnp.float32),
            pltpu.SemaphoreType.DMA,
            pltpu.SemaphoreType.DMA,
            pltpu.SemaphoreType.DMA,
            pltpu.SemaphoreType.DMA,
            pltpu.SemaphoreType.DMA,
        ],
    )
    return fn(ids_flat, tt_flat, word_emb, type_emb, pos_emb)


def kernel(input_ids, token_type_ids, word_emb, type_emb, pos_emb):
    ids_flat = input_ids.reshape(-1).astype(jnp.int32)
    tt_flat = token_type_ids.reshape(-1).astype(jnp.int32)
    out = _emb_call(ids_flat, tt_flat, word_emb, type_emb, pos_emb)
    return out.reshape(B, S, H)
